# baseline (speedup 1.0000x reference)
import functools

import jax
import jax.numpy as jnp
from jax import lax
from jax.experimental import pallas as pl
from jax.experimental.pallas import tpu as pltpu
from jax.experimental.pallas import tpu_sc as plsc

_info = plsc.get_sparse_core_info()
_NC, _NS = _info.num_cores, _info.num_subcores
_NW = _NC * _NS

_CHUNK = 128
_NBUF = 2


def _make_gather(batch, hist, dim):
  flat = batch * hist
  n_blocks = flat // (_NW * _CHUNK)
  blocks_per_h = batch // _CHUNK
  mesh = plsc.VectorSubcoreMesh(core_axis_name="c", subcore_axis_name="s")

  @functools.partial(
      pl.kernel,
      mesh=mesh,
      out_type=jax.ShapeDtypeStruct((hist, dim, batch), jnp.float32),
      scratch_types=[
          pltpu.VMEM((n_blocks, _CHUNK), jnp.int32),
          pltpu.VMEM((n_blocks, _CHUNK), jnp.int32),
          pltpu.VMEM((_NBUF, _CHUNK, 2 * dim), jnp.float32),
          pltpu.VMEM((_NBUF, dim, _CHUNK), jnp.float32),
      ]
      + [pltpu.SemaphoreType.DMA] * (2 * _NBUF),
      compiler_params=pltpu.CompilerParams(
          use_tc_tiling_on_sc=True, needs_layout_passes=False
      ),
  )
  def gather_kernel(idxhi_hbm, par_hbm, w2_hbm, out_hbm, idxhi_v, par_v,
                    wide_v, outb_v, *sems):
    gsem = sems[:_NBUF]
    ssem = sems[_NBUF:]
    wid = lax.axis_index("s") * _NC + lax.axis_index("c")
    pltpu.sync_copy(idxhi_hbm.at[wid], idxhi_v)
    pltpu.sync_copy(par_hbm.at[wid], par_v)

    for b in range(_NBUF):
      pltpu.async_copy(w2_hbm.at[idxhi_v.at[b]], wide_v.at[b], gsem[b])

    def outer(g, carry):
      for b in range(_NBUF):
        t = g * _NBUF + b
        pltpu.make_async_copy(
            w2_hbm.at[idxhi_v.at[t]], wide_v.at[b], gsem[b]
        ).wait()

        @pl.when(g > 0)
        def _():
          pltpu.make_async_copy(
              outb_v.at[b], out_hbm.at[0, :, pl.ds(0, _CHUNK)], ssem[b]
          ).wait()

        for jg in range(_CHUNK // 16):
          j_vec = jax.lax.broadcasted_iota(jnp.int32, (16,), 0) + jg * 16
          base = par_v[t, pl.ds(jg * 16, 16)]
          for d in range(dim):
            vals = plsc.load_gather(wide_v.at[b], [j_vec, base + d])
            outb_v[b, d, pl.ds(jg * 16, 16)] = vals

        blk = wid * n_blocks + t
        h = blk // blocks_per_h
        bb = blk % blocks_per_h
        pltpu.async_copy(
            outb_v.at[b], out_hbm.at[h, :, pl.ds(bb * _CHUNK, _CHUNK)],
            ssem[b],
        )

        @pl.when(t + _NBUF < n_blocks)
        def _():
          pltpu.async_copy(
              w2_hbm.at[idxhi_v.at[t + _NBUF]], wide_v.at[b], gsem[b]
          )

      return carry

    lax.fori_loop(0, n_blocks // _NBUF, outer, 0)

    for b in range(_NBUF):
      pltpu.make_async_copy(
          outb_v.at[b], out_hbm.at[0, :, pl.ds(0, _CHUNK)], ssem[b]
      ).wait()

  return gather_kernel


def kernel(input, weight):
  batch, hist = input.shape
  vocab, dim = weight.shape
  w2 = weight.reshape(vocab // 2, 2 * dim)
  idx = input.T.reshape(_NW, -1, _CHUNK).astype(jnp.int32)
  idx_hi = idx >> 1
  par64 = (idx & 1) * dim
  out3 = _make_gather(batch, hist, dim)(idx_hi, par64, w2)
  return out3.transpose(2, 0, 1)


# speedup vs baseline: 1.0654x; 1.0654x over previous
import functools

import jax
import jax.numpy as jnp
from jax import lax
from jax.experimental import pallas as pl
from jax.experimental.pallas import tpu as pltpu
from jax.experimental.pallas import tpu_sc as plsc

_info = plsc.get_sparse_core_info()
_NC, _NS = _info.num_cores, _info.num_subcores
_NW = _NC * _NS

_CHUNK = 128
_NBUF = 2


def _make_gather(batch, hist, dim):
  flat = batch * hist
  n_blocks = flat // (_NW * _CHUNK)
  blocks_per_h = batch // _CHUNK
  mesh = plsc.VectorSubcoreMesh(core_axis_name="c", subcore_axis_name="s")

  @functools.partial(
      pl.kernel,
      mesh=mesh,
      out_type=jax.ShapeDtypeStruct((hist, dim, batch), jnp.float32),
      scratch_types=[
          pltpu.VMEM((n_blocks, _CHUNK), jnp.int32),
          pltpu.VMEM((n_blocks, _CHUNK), jnp.int32),
          pltpu.VMEM((_NBUF, _CHUNK, 2 * dim), jnp.float32),
          pltpu.VMEM((_NBUF * dim, _CHUNK), jnp.float32),
      ]
      + [pltpu.SemaphoreType.DMA] * (2 * _NBUF),
      compiler_params=pltpu.CompilerParams(
          use_tc_tiling_on_sc=True, needs_layout_passes=False
      ),
  )
  def gather_kernel(idxhi_hbm, par_hbm, w2_hbm, out_hbm, idxhi_v, par_v,
                    wide_v, outb_v, *sems):
    gsem = sems[:_NBUF]
    ssem = sems[_NBUF:]
    wid = lax.axis_index("s") * _NC + lax.axis_index("c")
    pltpu.sync_copy(idxhi_hbm.at[wid], idxhi_v)
    pltpu.sync_copy(par_hbm.at[wid], par_v)

    for b in range(_NBUF):
      pltpu.async_copy(w2_hbm.at[idxhi_v.at[b]], wide_v.at[b], gsem[b])

    def outer(g, carry):
      for b in range(_NBUF):
        t = g * _NBUF + b
        pltpu.make_async_copy(
            w2_hbm.at[idxhi_v.at[t]], wide_v.at[b], gsem[b]
        ).wait()

        @pl.when(g > 0)
        def _():
          pltpu.make_async_copy(
              outb_v.at[pl.ds(b * dim, dim), :], out_hbm.at[0, :, pl.ds(0, _CHUNK)],
              ssem[b]
          ).wait()

        row_iota = jax.lax.broadcasted_iota(jnp.int32, (16,), 0)
        col_j = row_iota * 0
        for j in range(_CHUNK):
          sjv = plsc.load_gather(par_v.at[t], [col_j])
          base = sjv + row_iota
          for dg in range(dim // 16):
            vals = plsc.load_gather(wide_v.at[b, j], [base + dg * 16])
            plsc.store_scatter(
                outb_v.at[pl.ds(b * dim + dg * 16, 16), :],
                [row_iota, col_j],
                vals,
            )
          col_j = col_j + 1

        blk = wid * n_blocks + t
        h = blk // blocks_per_h
        bb = blk % blocks_per_h
        pltpu.async_copy(
            outb_v.at[pl.ds(b * dim, dim), :],
            out_hbm.at[h, :, pl.ds(bb * _CHUNK, _CHUNK)],
            ssem[b],
        )

        @pl.when(t + _NBUF < n_blocks)
        def _():
          pltpu.async_copy(
              w2_hbm.at[idxhi_v.at[t + _NBUF]], wide_v.at[b], gsem[b]
          )

      return carry

    lax.fori_loop(0, n_blocks // _NBUF, outer, 0)

    for b in range(_NBUF):
      pltpu.make_async_copy(
          outb_v.at[pl.ds(b * dim, dim), :], out_hbm.at[0, :, pl.ds(0, _CHUNK)],
              ssem[b]
      ).wait()

  return gather_kernel


def kernel(input, weight):
  batch, hist = input.shape
  vocab, dim = weight.shape
  w2 = weight.reshape(vocab // 2, 2 * dim)
  idx = input.T.reshape(_NW, -1, _CHUNK).astype(jnp.int32)
  idx_hi = idx >> 1
  par64 = (idx & 1) * dim
  out3 = _make_gather(batch, hist, dim)(idx_hi, par64, w2)
  return out3.transpose(2, 0, 1)
